# baseline (speedup 1.0000x reference)
import functools
import math

import jax
import jax.numpy as jnp
from jax import lax
from jax.experimental import pallas as pl
from jax.experimental.pallas import tpu as pltpu
from jax.experimental.pallas import tpu_sc as plsc

L = 16
NC = 2
NS = 16
NW = NC * NS
ROWS = 64
COLS = 32768
ROWS_PER_W = ROWS // NW
UNROLL = 16
CHUNK = UNROLL * L
CS = 16384
CH_PER_ROW = COLS // CS
NCH = ROWS_PER_W * CH_PER_ROW
CSTEPS = CS // CHUNK


def _pair_merge(a, b):
    a1, a2 = a
    b1, b2 = b
    return (jnp.maximum(a1, b1),
            jnp.maximum(jnp.minimum(a1, b1), jnp.maximum(a2, b2)))


def _lane_gather(x, idx):
    dn = lax.GatherDimensionNumbers(
        offset_dims=(), collapsed_slice_dims=(0,), start_index_map=(0,))
    return lax.gather(x, idx[:, None], dn, slice_sizes=(1,),
                      mode=lax.GatherScatterMode.PROMISE_IN_BOUNDS)


def _sc_second_max(p):
    mesh = plsc.VectorSubcoreMesh(core_axis_name="c", subcore_axis_name="s")

    @functools.partial(
        pl.kernel,
        mesh=mesh,
        out_type=jax.ShapeDtypeStruct((NW, L), jnp.float32),
        scratch_types=[
            pltpu.VMEM((NCH, CS), jnp.float32),
            pltpu.VMEM((L,), jnp.float32),
            pltpu.SemaphoreType.DMA,
            pltpu.SemaphoreType.DMA,
            pltpu.SemaphoreType.DMA,
            pltpu.SemaphoreType.DMA,
        ],
    )
    def k(p_hbm, out_hbm, buf_v, res_v, sem0, sem1, sem2, sem3):
        wid = lax.axis_index("s") * NC + lax.axis_index("c")
        base = wid * ROWS_PER_W
        sems = (sem0, sem1, sem2, sem3)

        handles = []
        for c in range(NCH):
            r, cc = divmod(c, CH_PER_ROW)
            handles.append(pltpu.async_copy(
                p_hbm.at[base + r, pl.ds(cc * CS, CS)],
                buf_v.at[c], sems[c]))

        neg_inf = jnp.full((L,), -jnp.inf, jnp.float32)
        lane = lax.iota(jnp.int32, L)
        res = jnp.zeros((L,), jnp.float32)

        carry = (neg_inf,) * (2 * UNROLL)
        for c in range(NCH):
            handles[c].wait()
            b = c

            def body(i, cr):
                out = []
                for j in range(UNROLL):
                    m1, m2 = cr[2 * j], cr[2 * j + 1]
                    v = buf_v[b, pl.ds(i * CHUNK + j * L, L)]
                    out.append(jnp.maximum(m1, v))
                    out.append(jnp.maximum(m2, jnp.minimum(m1, v)))
                return tuple(out)

            carry = lax.fori_loop(0, CSTEPS, body, carry)

            if (c + 1) % CH_PER_ROW == 0:
                r = c // CH_PER_ROW
                pairs = [(carry[2 * j], carry[2 * j + 1])
                         for j in range(UNROLL)]
                while len(pairs) > 1:
                    pairs = [_pair_merge(pairs[t], pairs[t + 1])
                             for t in range(0, len(pairs), 2)]
                m1, m2 = pairs[0]
                for s in (8, 4, 2, 1):
                    idx = lane ^ s
                    m1, m2 = _pair_merge(
                        (m1, m2), (_lane_gather(m1, idx), _lane_gather(m2, idx)))
                res = jnp.where(lane == r, m2, res)
                carry = (neg_inf,) * (2 * UNROLL)

        res_v[...] = res
        pltpu.sync_copy(res_v, out_hbm.at[wid])

    return k(p)


def _tc_finish(xs, mu, sigma):
    def body(xs_ref, mu_ref, sigma_ref, out_ref):
        x = xs_ref[...]
        mu_v = mu_ref[0, 0]
        sigma2 = sigma_ref[0, 0] * sigma_ref[0, 0]
        coef = 1.0 / jnp.sqrt(jnp.float32(2.0 * math.pi) * sigma2)
        pdf = coef - coef * jnp.exp(-((x - mu_v) ** 2) / (2.0 * sigma2))
        col = lax.broadcasted_iota(jnp.int32, (NW, L), 1)
        term = jnp.where(col < ROWS_PER_W, pdf, 0.0)
        out_ref[0, 0] = jnp.sum(term) * jnp.float32(10.0 / ROWS)

    return pl.pallas_call(
        body,
        out_shape=jax.ShapeDtypeStruct((1, 1), jnp.float32),
        in_specs=[
            pl.BlockSpec(memory_space=pltpu.VMEM),
            pl.BlockSpec(memory_space=pltpu.SMEM),
            pl.BlockSpec(memory_space=pltpu.SMEM),
        ],
        out_specs=pl.BlockSpec(memory_space=pltpu.SMEM),
    )(xs, mu.reshape(1, 1), sigma.reshape(1, 1))


def kernel(p, mu, sigma):
    xs = _sc_second_max(p)
    loss = _tc_finish(xs, mu.astype(jnp.float32), sigma.astype(jnp.float32))
    return loss[0, 0]


# speedup vs baseline: 1.0097x; 1.0097x over previous
import functools
import math

import jax
import jax.numpy as jnp
from jax import lax
from jax.experimental import pallas as pl
from jax.experimental.pallas import tpu as pltpu
from jax.experimental.pallas import tpu_sc as plsc

L = 16
NC = 2
NS = 16
NW = NC * NS
ROWS = 64
COLS = 32768
ROWS_PER_W = ROWS // NW
UNROLL = 8
CHUNK = UNROLL * L
CS = 16384
CH_PER_ROW = COLS // CS
NCH = ROWS_PER_W * CH_PER_ROW
CSTEPS = CS // CHUNK


def _pair_merge(a, b):
    a1, a2 = a
    b1, b2 = b
    return (jnp.maximum(a1, b1),
            jnp.maximum(jnp.minimum(a1, b1), jnp.maximum(a2, b2)))


def _lane_gather(x, idx):
    dn = lax.GatherDimensionNumbers(
        offset_dims=(), collapsed_slice_dims=(0,), start_index_map=(0,))
    return lax.gather(x, idx[:, None], dn, slice_sizes=(1,),
                      mode=lax.GatherScatterMode.PROMISE_IN_BOUNDS)


def _sc_second_max(p):
    mesh = plsc.VectorSubcoreMesh(core_axis_name="c", subcore_axis_name="s")

    @functools.partial(
        pl.kernel,
        mesh=mesh,
        out_type=jax.ShapeDtypeStruct((NW, L), jnp.float32),
        scratch_types=[
            pltpu.VMEM((NCH, CS), jnp.float32),
            pltpu.VMEM((L,), jnp.float32),
            pltpu.SemaphoreType.DMA,
            pltpu.SemaphoreType.DMA,
            pltpu.SemaphoreType.DMA,
            pltpu.SemaphoreType.DMA,
        ],
    )
    def k(p_hbm, out_hbm, buf_v, res_v, sem0, sem1, sem2, sem3):
        wid = lax.axis_index("s") * NC + lax.axis_index("c")
        base = wid * ROWS_PER_W
        sems = (sem0, sem1, sem2, sem3)

        handles = []
        for c in range(NCH):
            r, cc = divmod(c, CH_PER_ROW)
            handles.append(pltpu.async_copy(
                p_hbm.at[base + r, pl.ds(cc * CS, CS)],
                buf_v.at[c], sems[c]))

        neg_inf = jnp.full((L,), -jnp.inf, jnp.float32)
        lane = lax.iota(jnp.int32, L)
        res = jnp.zeros((L,), jnp.float32)

        carry = (neg_inf,) * (2 * UNROLL)
        for c in range(NCH):
            handles[c].wait()
            b = c

            def body(i, cr):
                out = []
                for j in range(UNROLL):
                    m1, m2 = cr[2 * j], cr[2 * j + 1]
                    v = buf_v[b, pl.ds(i * CHUNK + j * L, L)]
                    out.append(jnp.maximum(m1, v))
                    out.append(jnp.maximum(m2, jnp.minimum(m1, v)))
                return tuple(out)

            carry = plsc.parallel_loop(0, CSTEPS, 1, unroll=2,
                                       carry=carry)(body)

            if (c + 1) % CH_PER_ROW == 0:
                r = c // CH_PER_ROW
                pairs = [(carry[2 * j], carry[2 * j + 1])
                         for j in range(UNROLL)]
                while len(pairs) > 1:
                    pairs = [_pair_merge(pairs[t], pairs[t + 1])
                             for t in range(0, len(pairs), 2)]
                m1, m2 = pairs[0]
                for s in (8, 4, 2, 1):
                    idx = lane ^ s
                    m1, m2 = _pair_merge(
                        (m1, m2), (_lane_gather(m1, idx), _lane_gather(m2, idx)))
                res = jnp.where(lane == r, m2, res)
                carry = (neg_inf,) * (2 * UNROLL)

        res_v[...] = res
        pltpu.sync_copy(res_v, out_hbm.at[wid])

    return k(p)


def _tc_finish(xs, mu, sigma):
    def body(xs_ref, mu_ref, sigma_ref, out_ref):
        x = xs_ref[...]
        mu_v = mu_ref[0, 0]
        sigma2 = sigma_ref[0, 0] * sigma_ref[0, 0]
        coef = 1.0 / jnp.sqrt(jnp.float32(2.0 * math.pi) * sigma2)
        pdf = coef - coef * jnp.exp(-((x - mu_v) ** 2) / (2.0 * sigma2))
        col = lax.broadcasted_iota(jnp.int32, (NW, L), 1)
        term = jnp.where(col < ROWS_PER_W, pdf, 0.0)
        out_ref[0, 0] = jnp.sum(term) * jnp.float32(10.0 / ROWS)

    return pl.pallas_call(
        body,
        out_shape=jax.ShapeDtypeStruct((1, 1), jnp.float32),
        in_specs=[
            pl.BlockSpec(memory_space=pltpu.VMEM),
            pl.BlockSpec(memory_space=pltpu.SMEM),
            pl.BlockSpec(memory_space=pltpu.SMEM),
        ],
        out_specs=pl.BlockSpec(memory_space=pltpu.SMEM),
    )(xs, mu.reshape(1, 1), sigma.reshape(1, 1))


def kernel(p, mu, sigma):
    xs = _sc_second_max(p)
    loss = _tc_finish(xs, mu.astype(jnp.float32), sigma.astype(jnp.float32))
    return loss[0, 0]


# speedup vs baseline: 1.0127x; 1.0030x over previous
import functools
import math

import jax
import jax.numpy as jnp
from jax import lax
from jax.experimental import pallas as pl
from jax.experimental.pallas import tpu as pltpu
from jax.experimental.pallas import tpu_sc as plsc

L = 16
NC = 2
NS = 16
NW = NC * NS
ROWS = 64
COLS = 32768
ROWS_PER_W = ROWS // NW
UNROLL = 8
CHUNK = UNROLL * L
CS = 16384
CH_PER_ROW = COLS // CS
NCH = ROWS_PER_W * CH_PER_ROW
CSTEPS = CS // CHUNK


def _pair_merge(a, b):
    a1, a2 = a
    b1, b2 = b
    return (jnp.maximum(a1, b1),
            jnp.maximum(jnp.minimum(a1, b1), jnp.maximum(a2, b2)))


def _lane_gather(x, idx):
    dn = lax.GatherDimensionNumbers(
        offset_dims=(), collapsed_slice_dims=(0,), start_index_map=(0,))
    return lax.gather(x, idx[:, None], dn, slice_sizes=(1,),
                      mode=lax.GatherScatterMode.PROMISE_IN_BOUNDS)


def _sc_second_max(p):
    mesh = plsc.VectorSubcoreMesh(core_axis_name="c", subcore_axis_name="s")

    @functools.partial(
        pl.kernel,
        mesh=mesh,
        out_type=jax.ShapeDtypeStruct((NW, L), jnp.float32),
        scratch_types=[
            pltpu.VMEM((NCH, CS), jnp.float32),
            pltpu.VMEM((L,), jnp.float32),
            pltpu.SemaphoreType.DMA,
            pltpu.SemaphoreType.DMA,
            pltpu.SemaphoreType.DMA,
            pltpu.SemaphoreType.DMA,
        ],
    )
    def k(p_hbm, out_hbm, buf_v, res_v, sem0, sem1, sem2, sem3):
        wid = lax.axis_index("s") * NC + lax.axis_index("c")
        base = wid * ROWS_PER_W
        sems = (sem0, sem1, sem2, sem3)

        handles = []
        for c in range(NCH):
            r, cc = divmod(c, CH_PER_ROW)
            handles.append(pltpu.async_copy(
                p_hbm.at[base + r, pl.ds(cc * CS, CS)],
                buf_v.at[c], sems[c]))

        neg_inf = jnp.full((L,), -jnp.inf, jnp.float32)
        lane = lax.iota(jnp.int32, L)
        res = jnp.zeros((L,), jnp.float32)

        carry = (neg_inf,) * (2 * UNROLL)
        for c in range(NCH):
            handles[c].wait()
            b = c

            def body(i, cr):
                out = []
                for j in range(UNROLL):
                    m1, m2 = cr[2 * j], cr[2 * j + 1]
                    v = buf_v[b, pl.ds(i * CHUNK + j * L, L)]
                    out.append(jnp.maximum(m1, v))
                    out.append(jnp.maximum(m2, jnp.minimum(m1, v)))
                return tuple(out)

            carry = plsc.parallel_loop(0, CSTEPS, 1, unroll=2,
                                       carry=carry)(body)

            if (c + 1) % CH_PER_ROW == 0:
                r = c // CH_PER_ROW
                pairs = [(carry[2 * j], carry[2 * j + 1])
                         for j in range(UNROLL)]
                while len(pairs) > 1:
                    pairs = [_pair_merge(pairs[t], pairs[t + 1])
                             for t in range(0, len(pairs), 2)]
                m1, m2 = pairs[0]
                for s in (8, 4, 2, 1):
                    idx = lane ^ s
                    m1, m2 = _pair_merge(
                        (m1, m2), (_lane_gather(m1, idx), _lane_gather(m2, idx)))
                res = jnp.where(lane == r, m2, res)
                carry = (neg_inf,) * (2 * UNROLL)

        res_v[...] = res
        pltpu.sync_copy(res_v, out_hbm.at[wid])

    return k(p)


def _tc_finish(xs, mu, sigma):
    def body(xs_ref, mu_ref, sigma_ref, out_ref):
        x = xs_ref[...]
        mu_v = mu_ref[0]
        sigma2 = sigma_ref[0] * sigma_ref[0]
        coef = 1.0 / jnp.sqrt(jnp.float32(2.0 * math.pi) * sigma2)
        pdf = coef - coef * jnp.exp(-((x - mu_v) ** 2) / (2.0 * sigma2))
        col = lax.broadcasted_iota(jnp.int32, (NW, L), 1)
        term = jnp.where(col < ROWS_PER_W, pdf, 0.0)
        out_ref[0] = jnp.sum(term) * jnp.float32(10.0 / ROWS)

    return pl.pallas_call(
        body,
        out_shape=jax.ShapeDtypeStruct((1,), jnp.float32),
        in_specs=[
            pl.BlockSpec(memory_space=pltpu.VMEM),
            pl.BlockSpec(memory_space=pltpu.SMEM),
            pl.BlockSpec(memory_space=pltpu.SMEM),
        ],
        out_specs=pl.BlockSpec(memory_space=pltpu.SMEM),
    )(xs, mu, sigma)


def kernel(p, mu, sigma):
    xs = _sc_second_max(p)
    return _tc_finish(xs, mu, sigma)[0]
